# Initial kernel scaffold; baseline (speedup 1.0000x reference)
import jax
import jax.numpy as jnp
from jax.experimental import pallas as pl


def kernel(action, holding, dominos):
    raise NotImplementedError("write your pallas kernel here")


# speedup vs baseline: 1.7961x; 1.7961x over previous
import functools

import jax
import jax.numpy as jnp
from jax import lax
from jax.experimental import pallas as pl
from jax.experimental.pallas import tpu as pltpu
from jax.experimental.pallas import tpu_sc as plsc

C = 4096
NC, NS, L = 2, 16, 16
NW = NC * NS
W = C // NW
NG = W // L
R = 128
NB = C // R


def _body(act_hbm, hold_hbm, dom_hbm, outdom_hbm, outhold_hbm,
          act_v, hold_v, nh_v, dbuf, obuf):
    wid = lax.axis_index("s") * NC + lax.axis_index("c")
    n0 = wid * W

    pltpu.sync_copy(act_hbm.at[pl.ds(n0, W)], act_v)
    pltpu.sync_copy(hold_hbm.at[:], hold_v.at[pl.ds(0, C)])

    a_g = [act_v[pl.ds(g * L, L)] for g in range(NG)]
    A_g = [1.0 - a for a in a_g]

    zero = jnp.zeros((L,), jnp.float32)
    carry0 = tuple(zero for _ in range(3 * NG))

    def block_body(blk, carry):
        m0 = blk * R
        pltpu.sync_copy(dom_hbm.at[pl.ds(m0, R), pl.ds(n0, W)], dbuf)

        def row_body(m, t):
            h = hold_v[pl.ds(m0 + m, L)][0]
            hv = jnp.full((L,), h, jnp.float32)
            Hv = 1.0 - hv
            t = list(t)
            for g in range(NG):
                d = dbuf[m, pl.ds(g * L, L)]
                p1 = d * A_g[g]
                p2 = d * Hv
                obuf[m, pl.ds(g * L, L)] = p1 + p2 - p1 * p2
                pr = d * hv
                t0, t1, t2 = t[3 * g], t[3 * g + 1], t[3 * g + 2]
                n0v = jnp.maximum(t0, pr)
                r1 = jnp.minimum(t0, pr)
                n1v = jnp.maximum(t1, r1)
                r2 = jnp.minimum(t1, r1)
                n2v = jnp.maximum(t2, r2)
                t[3 * g], t[3 * g + 1], t[3 * g + 2] = n0v, n1v, n2v
            return tuple(t)

        carry = lax.fori_loop(0, R, row_body, carry)
        pltpu.sync_copy(obuf, outdom_hbm.at[pl.ds(m0, R), pl.ds(n0, W)])
        return carry

    carry = lax.fori_loop(0, NB, block_body, carry0)

    for g in range(NG):
        v0 = carry[3 * g] * a_g[g]
        v1 = carry[3 * g + 1] * a_g[g]
        v2 = carry[3 * g + 2] * a_g[g]
        nh_v[pl.ds(g * L, L)] = 1.0 - (1.0 - v0) * (1.0 - v1) * (1.0 - v2)
    pltpu.sync_copy(nh_v, outhold_hbm.at[pl.ds(n0, W)])


_sc_call = functools.partial(
    pl.kernel,
    out_type=[
        jax.ShapeDtypeStruct((C, C), jnp.float32),
        jax.ShapeDtypeStruct((C,), jnp.float32),
    ],
    mesh=plsc.VectorSubcoreMesh(
        core_axis_name="c", subcore_axis_name="s", num_cores=NC,
        num_subcores=NS),
    scratch_types=[
        pltpu.VMEM((W,), jnp.float32),
        pltpu.VMEM((C + L,), jnp.float32),
        pltpu.VMEM((W,), jnp.float32),
        pltpu.VMEM((R, W), jnp.float32),
        pltpu.VMEM((R, W), jnp.float32),
    ],
)(_body)


def kernel(action, holding, dominos):
    dom = dominos.reshape(C, C)
    out_dom, out_hold = _sc_call(action, holding, dom)
    return out_hold, out_dom.reshape(-1)


# speedup vs baseline: 2.2314x; 1.2424x over previous
import functools

import jax
import jax.numpy as jnp
from jax import lax
from jax.experimental import pallas as pl
from jax.experimental.pallas import tpu as pltpu
from jax.experimental.pallas import tpu_sc as plsc

C = 4096
NC, NS, L = 2, 16, 16
NW = NC * NS
W = C // NW
NG = W // L
R = 128
NB = C // R


def _body(act_hbm, hold_hbm, dom_hbm, outdom_hbm, outhold_hbm,
          act_v, hold_v, nh_v, dbuf, obuf,
          sem_in0, sem_in1, sem_out0, sem_out1):
    wid = lax.axis_index("s") * NC + lax.axis_index("c")
    n0 = wid * W
    sem_in = (sem_in0, sem_in1)
    sem_out = (sem_out0, sem_out1)

    pltpu.sync_copy(act_hbm.at[pl.ds(n0, W)], act_v)
    pltpu.sync_copy(hold_hbm.at[:], hold_v.at[pl.ds(0, C)])

    a_g = [act_v[pl.ds(g * L, L)] for g in range(NG)]
    A_g = [1.0 - a for a in a_g]

    zero = jnp.zeros((L,), jnp.float32)
    carry = tuple(zero for _ in range(3 * NG))

    def start_in(j):
        p = j % 2
        return pltpu.async_copy(
            dom_hbm.at[pl.ds(j * R, R), pl.ds(n0, W)], dbuf.at[p], sem_in[p])

    copies_in = {0: start_in(0), 1: start_in(1)}
    copies_out = {}

    for j in range(NB):
        p = j % 2
        db = dbuf.at[p]
        ob = obuf.at[p]
        copies_in[j].wait()
        if j >= 2:
            copies_out[j - 2].wait()
        m0 = j * R

        def row_body(m, t, db=db, ob=ob, m0=m0):
            h = hold_v[pl.ds(m0 + m, L)][0]
            hv = jnp.full((L,), h, jnp.float32)
            Hv = 1.0 - hv
            t = list(t)
            for g in range(NG):
                d = db[m, pl.ds(g * L, L)]
                p1 = d * A_g[g]
                p2 = d * Hv
                ob[m, pl.ds(g * L, L)] = p1 + p2 - p1 * p2
                pr = d * hv
                t0, t1, t2 = t[3 * g], t[3 * g + 1], t[3 * g + 2]
                n0v = jnp.maximum(t0, pr)
                r1 = jnp.minimum(t0, pr)
                n1v = jnp.maximum(t1, r1)
                r2 = jnp.minimum(t1, r1)
                n2v = jnp.maximum(t2, r2)
                t[3 * g], t[3 * g + 1], t[3 * g + 2] = n0v, n1v, n2v
            return tuple(t)

        carry = lax.fori_loop(0, R, row_body, carry)
        copies_out[j] = pltpu.async_copy(
            ob, outdom_hbm.at[pl.ds(m0, R), pl.ds(n0, W)], sem_out[p])
        if j + 2 < NB:
            copies_in[j + 2] = start_in(j + 2)

    copies_out[NB - 2].wait()
    copies_out[NB - 1].wait()

    for g in range(NG):
        v0 = carry[3 * g] * a_g[g]
        v1 = carry[3 * g + 1] * a_g[g]
        v2 = carry[3 * g + 2] * a_g[g]
        nh_v[pl.ds(g * L, L)] = 1.0 - (1.0 - v0) * (1.0 - v1) * (1.0 - v2)
    pltpu.sync_copy(nh_v, outhold_hbm.at[pl.ds(n0, W)])


_sc_call = functools.partial(
    pl.kernel,
    out_type=[
        jax.ShapeDtypeStruct((C, C), jnp.float32),
        jax.ShapeDtypeStruct((C,), jnp.float32),
    ],
    mesh=plsc.VectorSubcoreMesh(
        core_axis_name="c", subcore_axis_name="s", num_cores=NC,
        num_subcores=NS),
    scratch_types=[
        pltpu.VMEM((W,), jnp.float32),
        pltpu.VMEM((C + L,), jnp.float32),
        pltpu.VMEM((W,), jnp.float32),
        pltpu.VMEM((2, R, W), jnp.float32),
        pltpu.VMEM((2, R, W), jnp.float32),
        pltpu.SemaphoreType.DMA,
        pltpu.SemaphoreType.DMA,
        pltpu.SemaphoreType.DMA,
        pltpu.SemaphoreType.DMA,
    ],
)(_body)


def kernel(action, holding, dominos):
    dom = dominos.reshape(C, C)
    out_dom, out_hold = _sc_call(action, holding, dom)
    return out_hold, out_dom.reshape(-1)
